# baseline (device time: 26471 ns/iter reference)
import jax
import jax.numpy as jnp
from jax import lax
from jax.experimental import pallas as pl
from jax.experimental.pallas import tpu as pltpu

N_DEV = 8
N_CHUNKS = 8
SCANS_BEFORE_WAIT = 3


def kernel(x):
    m, n = x.shape
    R = m // N_CHUNKS

    def body(
        x_hbm,
        out_hbm,
        xbuf,
        sbuf,
        t_ref,
        comm_ref,
        load_sems,
        store_sems,
        send_sems,
        recv_sems,
    ):
        my_i = lax.axis_index("i")

        barrier_sem = pltpu.get_barrier_semaphore()
        for k in range(1, N_DEV):
            peer = (my_i + k) % N_DEV
            pl.semaphore_signal(
                barrier_sem,
                inc=1,
                device_id=(peer,),
                device_id_type=pl.DeviceIdType.MESH,
            )

        def load(c):
            return pltpu.make_async_copy(
                x_hbm.at[pl.ds(c * R, R), :],
                xbuf.at[pl.ds(c * R, R), :],
                load_sems.at[c],
            )

        for c in range(N_CHUNKS):
            load(c).start()

        g = [None] * N_CHUNKS
        carry = None
        for c in range(N_CHUNKS):
            load(c).wait()
            t = xbuf[pl.ds(c * R, R), :]
            h = R
            while h > 1:
                h //= 2
                t = t[:h, :] * t[h:, :]
            g[c] = carry
            carry = t if carry is None else carry * t
        t_ref[...] = carry

        pl.semaphore_wait(barrier_sem, N_DEV - 1)
        rdmas = []
        for k in range(1, N_DEV):
            dst = (my_i + k) % N_DEV
            rdma = pltpu.make_async_remote_copy(
                src_ref=t_ref,
                dst_ref=comm_ref.at[pl.ds(k - 1, 1)],
                send_sem=send_sems.at[k - 1],
                recv_sem=recv_sems.at[k - 1],
                device_id=(dst,),
                device_id_type=pl.DeviceIdType.MESH,
            )
            rdma.start()
            rdmas.append(rdma)

        def scan(c):
            a = xbuf[pl.ds(c * R, R), :]
            d = 1
            while d < R:
                shifted = jnp.concatenate(
                    [jnp.ones((d, n), jnp.float32), a[:-d, :]], axis=0
                )
                a = a * shifted
                d *= 2
            return a

        scanned = {}
        for c in range(SCANS_BEFORE_WAIT):
            scanned[c] = scan(c)

        for rdma in rdmas:
            rdma.wait()

        comm = comm_ref[...]
        kvec = lax.broadcasted_iota(jnp.int32, (N_DEV - 1, n), 0) + 1
        vals = jnp.where(kvec <= my_i, comm, jnp.ones_like(comm))
        prefix = vals[0:1, :]
        for k in range(1, N_DEV - 1):
            prefix = prefix * vals[k : k + 1, :]

        coef = [prefix if g[c] is None else g[c] * prefix for c in range(N_CHUNKS)]

        def store(c, slot):
            return pltpu.make_async_copy(
                sbuf.at[slot],
                out_hbm.at[pl.ds(c * R, R), :],
                store_sems.at[slot],
            )

        for c in range(N_CHUNKS):
            slot = c % 2
            a = scanned[c] if c in scanned else scan(c)
            if c >= 2:
                store(c - 2, slot).wait()
            sbuf[slot] = (a * coef[c]).astype(jnp.bfloat16)
            store(c, slot).start()
        store(N_CHUNKS - 2, N_CHUNKS % 2).wait()
        store(N_CHUNKS - 1, (N_CHUNKS + 1) % 2).wait()

    return pl.pallas_call(
        body,
        out_shape=jax.ShapeDtypeStruct((m, n), jnp.bfloat16),
        in_specs=[pl.BlockSpec(memory_space=pl.ANY)],
        out_specs=pl.BlockSpec(memory_space=pl.ANY),
        scratch_shapes=[
            pltpu.VMEM((m, n), jnp.float32),
            pltpu.VMEM((2, R, n), jnp.bfloat16),
            pltpu.VMEM((1, n), jnp.float32),
            pltpu.VMEM((N_DEV - 1, n), jnp.float32),
            pltpu.SemaphoreType.DMA((N_CHUNKS,)),
            pltpu.SemaphoreType.DMA((2,)),
            pltpu.SemaphoreType.DMA((N_DEV - 1,)),
            pltpu.SemaphoreType.DMA((N_DEV - 1,)),
        ],
        compiler_params=pltpu.CompilerParams(collective_id=0),
    )(x)


# device time: 19431 ns/iter; 1.3623x vs baseline; 1.3623x over previous
import jax
import jax.numpy as jnp
from jax import lax
from jax.experimental import pallas as pl
from jax.experimental.pallas import tpu as pltpu

N_DEV = 8
N_LOADS = 8
SUBS_PER_LOAD = 2
N_SUBS = N_LOADS * SUBS_PER_LOAD
LOADS_WITH_SCANS = 4


def kernel(x):
    m, n = x.shape
    RL = m // N_LOADS
    RS = m // N_SUBS

    def body(
        x_hbm,
        out_hbm,
        xbuf,
        lbuf,
        sbuf,
        t_ref,
        comm_ref,
        load_sems,
        store_sems,
        send_sems,
        recv_sems,
    ):
        my_i = lax.axis_index("i")

        barrier_sem = pltpu.get_barrier_semaphore()
        for k in range(1, N_DEV):
            peer = (my_i + k) % N_DEV
            pl.semaphore_signal(
                barrier_sem,
                inc=1,
                device_id=(peer,),
                device_id_type=pl.DeviceIdType.MESH,
            )

        def load(c):
            return pltpu.make_async_copy(
                x_hbm.at[pl.ds(c * RL, RL), :],
                xbuf.at[pl.ds(c * RL, RL), :],
                load_sems.at[c],
            )

        for c in range(N_LOADS):
            load(c).start()

        def scan_to_lbuf(s):
            a = xbuf[pl.ds(s * RS, RS), :]
            d = 1
            while d < RS:
                shifted = jnp.concatenate(
                    [jnp.ones((d, n), jnp.float32), a[:-d, :]], axis=0
                )
                a = a * shifted
                d *= 2
            lbuf[pl.ds(s * RS, RS), :] = a.astype(jnp.bfloat16)

        g = [None] * N_SUBS
        carry = None
        for c in range(N_LOADS):
            load(c).wait()
            for j in range(SUBS_PER_LOAD):
                s = c * SUBS_PER_LOAD + j
                t = xbuf[pl.ds(s * RS, RS), :]
                h = RS
                while h > 1:
                    h //= 2
                    t = t[:h, :] * t[h:, :]
                g[s] = carry
                carry = t if carry is None else carry * t
            if 1 <= c <= LOADS_WITH_SCANS:
                scan_to_lbuf((c - 1) * SUBS_PER_LOAD)
                scan_to_lbuf((c - 1) * SUBS_PER_LOAD + 1)
        t_ref[...] = carry

        pl.semaphore_wait(barrier_sem, N_DEV - 1)
        rdmas = []
        for k in range(1, N_DEV):
            dst = (my_i + k) % N_DEV
            rdma = pltpu.make_async_remote_copy(
                src_ref=t_ref,
                dst_ref=comm_ref.at[pl.ds(k - 1, 1)],
                send_sem=send_sems.at[k - 1],
                recv_sem=recv_sems.at[k - 1],
                device_id=(dst,),
                device_id_type=pl.DeviceIdType.MESH,
            )
            rdma.start()
            rdmas.append(rdma)

        for s in range(LOADS_WITH_SCANS * SUBS_PER_LOAD, N_SUBS):
            scan_to_lbuf(s)

        for rdma in rdmas:
            rdma.wait()

        comm = comm_ref[...]
        kvec = lax.broadcasted_iota(jnp.int32, (N_DEV - 1, n), 0) + 1
        vals = jnp.where(kvec <= my_i, comm, jnp.ones_like(comm))
        prefix = vals[0:1, :]
        for k in range(1, N_DEV - 1):
            prefix = prefix * vals[k : k + 1, :]

        coef = [prefix if g[s] is None else g[s] * prefix for s in range(N_SUBS)]

        def store(s, slot):
            return pltpu.make_async_copy(
                sbuf.at[slot],
                out_hbm.at[pl.ds(s * RS, RS), :],
                store_sems.at[slot],
            )

        for s in range(N_SUBS):
            slot = s % 2
            if s >= 2:
                store(s - 2, slot).wait()
            sbuf[slot] = (
                lbuf[pl.ds(s * RS, RS), :] * coef[s]
            ).astype(jnp.bfloat16)
            store(s, slot).start()
        store(N_SUBS - 2, N_SUBS % 2).wait()
        store(N_SUBS - 1, (N_SUBS + 1) % 2).wait()

    return pl.pallas_call(
        body,
        out_shape=jax.ShapeDtypeStruct((m, n), jnp.bfloat16),
        in_specs=[pl.BlockSpec(memory_space=pl.ANY)],
        out_specs=pl.BlockSpec(memory_space=pl.ANY),
        scratch_shapes=[
            pltpu.VMEM((m, n), jnp.float32),
            pltpu.VMEM((m, n), jnp.bfloat16),
            pltpu.VMEM((2, RS, n), jnp.bfloat16),
            pltpu.VMEM((1, n), jnp.float32),
            pltpu.VMEM((N_DEV - 1, n), jnp.float32),
            pltpu.SemaphoreType.DMA((N_LOADS,)),
            pltpu.SemaphoreType.DMA((2,)),
            pltpu.SemaphoreType.DMA((N_DEV - 1,)),
            pltpu.SemaphoreType.DMA((N_DEV - 1,)),
        ],
        compiler_params=pltpu.CompilerParams(collective_id=0),
    )(x)


# device time: 19397 ns/iter; 1.3647x vs baseline; 1.0018x over previous
import jax
import jax.numpy as jnp
from jax import lax
from jax.experimental import pallas as pl
from jax.experimental.pallas import tpu as pltpu

N_DEV = 8
N_LOADS = 8
SUBS_PER_LOAD = 2
N_SUBS = N_LOADS * SUBS_PER_LOAD
LOADS_WITH_SCANS = 4


def kernel(x):
    m, n = x.shape
    RL = m // N_LOADS
    RS = m // N_SUBS

    def body(
        x_hbm,
        out_hbm,
        xbuf,
        lbuf,
        sbuf,
        t_ref,
        comm_ref,
        load_sems,
        store_sems,
        send_sems,
        recv_sems,
    ):
        my_i = lax.axis_index("i")

        barrier_sem = pltpu.get_barrier_semaphore()
        for k in range(1, N_DEV):
            peer = (my_i + k) % N_DEV
            pl.semaphore_signal(
                barrier_sem,
                inc=1,
                device_id=(peer,),
                device_id_type=pl.DeviceIdType.MESH,
            )

        def load(c):
            return pltpu.make_async_copy(
                x_hbm.at[pl.ds(c * RL, RL), :],
                xbuf.at[pl.ds(c * RL, RL), :],
                load_sems.at[c],
            )

        for c in range(N_LOADS):
            load(c).start()

        def scan(s):
            a = xbuf[pl.ds(s * RS, RS), :]
            d = 1
            while d < RS:
                shifted = jnp.concatenate(
                    [jnp.ones((d, n), jnp.float32), a[:-d, :]], axis=0
                )
                a = a * shifted
                d *= 2
            return a

        def scan_to_lbuf(s):
            lbuf[pl.ds(s * RS, RS), :] = scan(s).astype(jnp.bfloat16)

        g = [None] * N_SUBS
        carry = None
        for c in range(N_LOADS):
            load(c).wait()
            for j in range(SUBS_PER_LOAD):
                s = c * SUBS_PER_LOAD + j
                t = xbuf[pl.ds(s * RS, RS), :]
                h = RS
                while h > 1:
                    h //= 2
                    t = t[:h, :] * t[h:, :]
                g[s] = carry
                carry = t if carry is None else carry * t
            if 1 <= c <= LOADS_WITH_SCANS:
                scan_to_lbuf((c - 1) * SUBS_PER_LOAD)
                scan_to_lbuf((c - 1) * SUBS_PER_LOAD + 1)
        t_ref[...] = carry

        pl.semaphore_wait(barrier_sem, N_DEV - 1)
        rdmas = []
        for k in range(1, N_DEV):
            dst = (my_i + k) % N_DEV
            rdma = pltpu.make_async_remote_copy(
                src_ref=t_ref,
                dst_ref=comm_ref.at[pl.ds(k - 1, 1)],
                send_sem=send_sems.at[k - 1],
                recv_sem=recv_sems.at[k - 1],
                device_id=(dst,),
                device_id_type=pl.DeviceIdType.MESH,
            )
            rdma.start()
            rdmas.append(rdma)

        for s in range(LOADS_WITH_SCANS * SUBS_PER_LOAD, N_SUBS):
            scan_to_lbuf(s)

        for rdma in rdmas:
            rdma.wait()

        comm = comm_ref[...]
        kvec = lax.broadcasted_iota(jnp.int32, (N_DEV - 1, n), 0) + 1
        vals = jnp.where(kvec <= my_i, comm, jnp.ones_like(comm))
        prefix = vals[0:1, :]
        for k in range(1, N_DEV - 1):
            prefix = prefix * vals[k : k + 1, :]

        coef = [prefix if g[s] is None else g[s] * prefix for s in range(N_SUBS)]

        def store(s, slot):
            return pltpu.make_async_copy(
                sbuf.at[slot],
                out_hbm.at[pl.ds(s * RS, RS), :],
                store_sems.at[slot],
            )

        for s in range(N_SUBS):
            slot = s % 2
            if s >= 2:
                store(s - 2, slot).wait()
            sbuf[slot] = (
                lbuf[pl.ds(s * RS, RS), :] * coef[s]
            ).astype(jnp.bfloat16)
            store(s, slot).start()
        store(N_SUBS - 2, N_SUBS % 2).wait()
        store(N_SUBS - 1, (N_SUBS + 1) % 2).wait()

    return pl.pallas_call(
        body,
        out_shape=jax.ShapeDtypeStruct((m, n), jnp.bfloat16),
        in_specs=[pl.BlockSpec(memory_space=pl.ANY)],
        out_specs=pl.BlockSpec(memory_space=pl.ANY),
        scratch_shapes=[
            pltpu.VMEM((m, n), jnp.float32),
            pltpu.VMEM((m, n), jnp.bfloat16),
            pltpu.VMEM((2, RS, n), jnp.bfloat16),
            pltpu.VMEM((1, n), jnp.float32),
            pltpu.VMEM((N_DEV - 1, n), jnp.float32),
            pltpu.SemaphoreType.DMA((N_LOADS,)),
            pltpu.SemaphoreType.DMA((2,)),
            pltpu.SemaphoreType.DMA((N_DEV - 1,)),
            pltpu.SemaphoreType.DMA((N_DEV - 1,)),
        ],
        compiler_params=pltpu.CompilerParams(collective_id=0),
    )(x)
